# Initial kernel scaffold; baseline (speedup 1.0000x reference)
import jax
import jax.numpy as jnp
from jax.experimental import pallas as pl


def kernel(x, ast_edge_index, cfg_edge_index, ddg_edge_index, ncs_edge_index, W_edge, Wz, Uz, bz, Wr, Ur, br, Wh, Uh, bh, Wa1, ba1, Wa2, ba2, fca_w, fca_b, Wb1, bb1, Wb2, bb2, fcb_w, fcb_b):
    raise NotImplementedError("write your pallas kernel here")


# speedup vs baseline: 4.9516x; 4.9516x over previous
import functools

import jax
import jax.numpy as jnp
from jax import lax
from jax.experimental import pallas as pl
from jax.experimental.pallas import tpu as pltpu
from jax.experimental.pallas import tpu_sc as plsc

N = 10000
E = 320000
D = 128
EMB = 128
F1 = D + EMB
CC = 128
NC = 2

NUM_CORES = 2
NUM_SUB = 16
NW = NUM_CORES * NUM_SUB
K = 128
CHUNKS = E // K
BASE_CHUNKS = CHUNKS // NW
EXTRA = CHUNKS % NW
NP = 10240
ROWS_PER_TILE = NP // NUM_SUB

@functools.lru_cache(maxsize=1)
def _build_sc_segment_sum():
    mesh = plsc.VectorSubcoreMesh(core_axis_name="c", subcore_axis_name="s",
                                  num_cores=NUM_CORES, num_subcores=NUM_SUB)

    @functools.partial(
        pl.kernel,
        out_type=jax.ShapeDtypeStruct((NUM_CORES, NP, D), jnp.float32),
        mesh=mesh,
        scratch_types=[
            pltpu.VMEM((K,), jnp.int32),
            pltpu.VMEM((K,), jnp.int32),
            pltpu.VMEM((K, D), jnp.float32),
            pltpu.VMEM_SHARED((NP, D), jnp.float32),
            pltpu.SemaphoreType.DMA,
        ],
    )
    def _sc_segment_sum(hw_hbm, src_hbm, dst_hbm, zeros_hbm, out_hbm,
                        src_v, dst_v, rows_v, acc, sem):
        c = lax.axis_index("c")
        s = lax.axis_index("s")
        w = s * NUM_CORES + c
        rows0 = s * ROWS_PER_TILE
        pltpu.sync_copy(zeros_hbm, acc.at[pl.ds(rows0, ROWS_PER_TILE)])
        plsc.subcore_barrier()
        nchunks = BASE_CHUNKS + jnp.where(w < EXTRA, 1, 0)

        def body(j, carry):
            off = (j * NW + w) * K
            pltpu.sync_copy(src_hbm.at[pl.ds(off, K)], src_v)
            pltpu.sync_copy(dst_hbm.at[pl.ds(off, K)], dst_v)
            pltpu.async_copy(hw_hbm.at[src_v], rows_v, sem).wait()
            pltpu.sync_copy(rows_v, acc.at[dst_v], add=True)
            return carry

        lax.fori_loop(0, nchunks, body, 0)
        plsc.subcore_barrier()
        pltpu.sync_copy(acc.at[pl.ds(rows0, ROWS_PER_TILE)],
                        out_hbm.at[c, pl.ds(rows0, ROWS_PER_TILE)])

    return _sc_segment_sum


BR = 1000


def _mm_body(x_ref, w_ref, o_ref):
    o_ref[...] = jnp.dot(x_ref[...], w_ref[...],
                         preferred_element_type=jnp.float32)


_mm = pl.pallas_call(
    _mm_body,
    grid=(N // BR,),
    in_specs=[pl.BlockSpec((BR, D), lambda i: (i, 0)),
              pl.BlockSpec((D, D), lambda i: (0, 0))],
    out_specs=pl.BlockSpec((BR, D), lambda i: (i, 0)),
    out_shape=jax.ShapeDtypeStruct((N, D), jnp.float32),
)


def _gru_body(h_ref, m0_ref, m1_ref, Wz_ref, Uz_ref, Wr_ref, Ur_ref,
              Wh_ref, Uh_ref, b_ref, Wn_ref, ho_ref, hwo_ref):
    h = h_ref[...]
    m = m0_ref[...] + m1_ref[...]
    dot = functools.partial(jnp.dot, preferred_element_type=jnp.float32)
    z = jax.nn.sigmoid(dot(m, Wz_ref[...]) + dot(h, Uz_ref[...]) + b_ref[0])
    r = jax.nn.sigmoid(dot(m, Wr_ref[...]) + dot(h, Ur_ref[...]) + b_ref[1])
    hc = jnp.tanh(dot(m, Wh_ref[...]) + dot(r * h, Uh_ref[...]) + b_ref[2])
    hn = (1.0 - z) * h + z * hc
    ho_ref[...] = hn
    hwo_ref[...] = dot(hn, Wn_ref[...])


_gru_step = pl.pallas_call(
    _gru_body,
    grid=(N // BR,),
    in_specs=[pl.BlockSpec((BR, D), lambda i: (i, 0)),
              pl.BlockSpec((BR, D), lambda i: (i, 0)),
              pl.BlockSpec((BR, D), lambda i: (i, 0)),
              pl.BlockSpec((D, D), lambda i: (0, 0)),
              pl.BlockSpec((D, D), lambda i: (0, 0)),
              pl.BlockSpec((D, D), lambda i: (0, 0)),
              pl.BlockSpec((D, D), lambda i: (0, 0)),
              pl.BlockSpec((D, D), lambda i: (0, 0)),
              pl.BlockSpec((D, D), lambda i: (0, 0)),
              pl.BlockSpec((3, D), lambda i: (0, 0)),
              pl.BlockSpec((D, D), lambda i: (0, 0))],
    out_specs=[pl.BlockSpec((BR, D), lambda i: (i, 0)),
               pl.BlockSpec((BR, D), lambda i: (i, 0))],
    out_shape=[jax.ShapeDtypeStruct((N, D), jnp.float32),
               jax.ShapeDtypeStruct((N, D), jnp.float32)],
)


def _shift_down(a):
    return jnp.concatenate([jnp.zeros((1, a.shape[1]), a.dtype), a[:-1]], 0)


def _shift_up(a):
    return jnp.concatenate([a[1:], jnp.zeros((1, a.shape[1]), a.dtype)], 0)


def _conv3_in(a, W_ref, b_ref):
    dot = functools.partial(jnp.dot, preferred_element_type=jnp.float32)
    return (dot(_shift_down(a), W_ref[0]) + dot(a, W_ref[1])
            + dot(_shift_up(a), W_ref[2]) + b_ref[0])


def _conv1_body(h_ref, x_ref, Wa_ref, ba_ref, Wb_ref, bb_ref, ca_ref, cb_ref):
    h = h_ref[...]
    zc = jnp.concatenate([h, x_ref[...]], 1)
    ca_ref[...] = jnp.maximum(_conv3_in(zc, Wa_ref, ba_ref), 0.0)
    cb_ref[...] = jnp.maximum(_conv3_in(h, Wb_ref, bb_ref), 0.0)


_conv1 = pl.pallas_call(
    _conv1_body,
    out_shape=[jax.ShapeDtypeStruct((N, CC), jnp.float32),
               jax.ShapeDtypeStruct((N, CC), jnp.float32)],
)


def _conv2_body(ea_ref, oa_ref, eb_ref, ob_ref, Wa_ref, ba_ref,
                Wb_ref, bb_ref, ca_ref, cb_ref):
    oa = oa_ref[...]
    pa = jnp.maximum(jnp.maximum(ea_ref[...], oa), _shift_down(oa))
    ob = ob_ref[...]
    pb = jnp.maximum(jnp.maximum(eb_ref[...], ob), _shift_down(ob))
    ca_ref[...] = jnp.maximum(_conv3_in(pa, Wa_ref, ba_ref), 0.0)
    cb_ref[...] = jnp.maximum(_conv3_in(pb, Wb_ref, bb_ref), 0.0)


_conv2 = pl.pallas_call(
    _conv2_body,
    out_shape=[jax.ShapeDtypeStruct((N // 2, CC), jnp.float32),
               jax.ShapeDtypeStruct((N // 2, CC), jnp.float32)],
)


def _head_body(ea_ref, oa_ref, eb_ref, ob_ref, wa_ref, ba_ref,
               wb_ref, bb_ref, o_ref):
    dot = functools.partial(jnp.dot, preferred_element_type=jnp.float32)
    oa = oa_ref[...]
    pa = jnp.maximum(jnp.maximum(ea_ref[...], oa), _shift_down(oa))
    ob = ob_ref[...]
    pb = jnp.maximum(jnp.maximum(eb_ref[...], ob), _shift_down(ob))
    ya = dot(pa, wa_ref[...]) + ba_ref[0]
    yb = dot(pb, wb_ref[...]) + bb_ref[0]
    o_ref[...] = jnp.sum(ya * yb, axis=0, keepdims=True) * (1.0 / (N // 4))


_head = pl.pallas_call(
    _head_body,
    out_shape=jax.ShapeDtypeStruct((1, NC), jnp.float32),
)


def kernel(x, ast_edge_index, cfg_edge_index, ddg_edge_index, ncs_edge_index,
           W_edge, Wz, Uz, bz, Wr, Ur, br, Wh, Uh, bh,
           Wa1, ba1, Wa2, ba2, fca_w, fca_b,
           Wb1, bb1, Wb2, bb2, fcb_w, fcb_b):
    edges = [ast_edge_index, cfg_edge_index, ddg_edge_index, ncs_edge_index]
    zeros_tile = jnp.zeros((ROWS_PER_TILE, D), jnp.float32)
    b3 = jnp.stack([bz, br, bh])

    sc_segment_sum = _build_sc_segment_sum()
    h = x
    hw = _mm(h, W_edge[0])
    for t in range(4):
        parts = sc_segment_sum(hw, edges[t][0], edges[t][1], zeros_tile)
        h, hw = _gru_step(h, parts[0, :N], parts[1, :N], Wz, Uz, Wr, Ur, Wh,
                          Uh, b3,
                          W_edge[(t + 1) % 4])

    ca, cb = _conv1(h, x, Wa1, ba1.reshape(1, CC), Wb1, bb1.reshape(1, CC))
    c2a, c2b = _conv2(ca[0::2], ca[1::2], cb[0::2], cb[1::2],
                      Wa2, ba2.reshape(1, CC), Wb2, bb2.reshape(1, CC))
    y = _head(c2a[0::2], c2a[1::2], c2b[0::2], c2b[1::2],
              fca_w, fca_b.reshape(1, NC), fcb_w, fcb_b.reshape(1, NC))
    return y.reshape(NC)


# speedup vs baseline: 8.4822x; 1.7130x over previous
import functools

import jax
import jax.numpy as jnp
from jax import lax
from jax.experimental import pallas as pl
from jax.experimental.pallas import tpu as pltpu
from jax.experimental.pallas import tpu_sc as plsc

N = 10000
E = 320000
D = 128
EMB = 128
F1 = D + EMB
CC = 128
NC = 2

NUM_CORES = 2
NUM_SUB = 16
NW = NUM_CORES * NUM_SUB
EPT = E // NW
K = 80
NCH = EPT // K
NP = 10240
ROWS_PER_TILE = NP // NUM_SUB

@functools.lru_cache(maxsize=1)
def _build_sc_segment_sum():
    mesh = plsc.VectorSubcoreMesh(core_axis_name="c", subcore_axis_name="s",
                                  num_cores=NUM_CORES, num_subcores=NUM_SUB)

    @functools.partial(
        pl.kernel,
        out_type=jax.ShapeDtypeStruct((NUM_CORES, NP, D), jnp.float32),
        mesh=mesh,
        scratch_types=[
            pltpu.VMEM((EPT,), jnp.int32),
            pltpu.VMEM((EPT,), jnp.int32),
            pltpu.VMEM((K, D), jnp.float32),
            pltpu.VMEM((K, D), jnp.float32),
            pltpu.VMEM_SHARED((NP, D), jnp.float32),
            pltpu.SemaphoreType.DMA,
            pltpu.SemaphoreType.DMA,
            pltpu.SemaphoreType.DMA,
        ],
        compiler_params=pltpu.CompilerParams(use_tc_tiling_on_sc=False),
    )
    def _sc_segment_sum(hw_hbm, src_hbm, dst_hbm, zeros_hbm, out_hbm,
                        src_v, dst_v, rows0_v, rows1_v, acc,
                        gsem0, gsem1, zsem):
        c = lax.axis_index("c")
        s = lax.axis_index("s")
        w = s * NUM_CORES + c
        rows0 = s * ROWS_PER_TILE

        def sidx(j):
            return src_v.at[pl.ds(j * K, K)]

        def didx(j):
            return dst_v.at[pl.ds(j * K, K)]

        pltpu.async_copy(zeros_hbm, acc.at[pl.ds(rows0, ROWS_PER_TILE)], zsem)
        pltpu.sync_copy(src_hbm.at[pl.ds(w * EPT, EPT)], src_v)
        pltpu.sync_copy(dst_hbm.at[pl.ds(w * EPT, EPT)], dst_v)
        pltpu.async_copy(hw_hbm.at[sidx(0)], rows0_v, gsem0)
        pltpu.async_copy(hw_hbm.at[sidx(1)], rows1_v, gsem1)
        pltpu.make_async_copy(zeros_hbm,
                              acc.at[pl.ds(rows0, ROWS_PER_TILE)], zsem).wait()
        plsc.subcore_barrier()

        def body(jj, carry):
            j = jj * 2
            pltpu.make_async_copy(hw_hbm.at[sidx(j)], rows0_v, gsem0).wait()
            pltpu.sync_copy(rows0_v, acc.at[didx(j)], add=True)
            pltpu.async_copy(hw_hbm.at[sidx(j + 2)], rows0_v, gsem0)
            pltpu.make_async_copy(hw_hbm.at[sidx(j + 1)], rows1_v,
                                  gsem1).wait()
            pltpu.sync_copy(rows1_v, acc.at[didx(j + 1)], add=True)
            pltpu.async_copy(hw_hbm.at[sidx(j + 3)], rows1_v, gsem1)
            return carry

        lax.fori_loop(0, (NCH - 3) // 2, body, 0)
        pltpu.make_async_copy(hw_hbm.at[sidx(NCH - 3)], rows0_v, gsem0).wait()
        pltpu.sync_copy(rows0_v, acc.at[didx(NCH - 3)], add=True)
        pltpu.make_async_copy(hw_hbm.at[sidx(NCH - 2)], rows1_v, gsem1).wait()
        pltpu.sync_copy(rows1_v, acc.at[didx(NCH - 2)], add=True)
        pltpu.async_copy(hw_hbm.at[sidx(NCH - 1)], rows0_v, gsem0)
        pltpu.make_async_copy(hw_hbm.at[sidx(NCH - 1)], rows0_v, gsem0).wait()
        pltpu.sync_copy(rows0_v, acc.at[didx(NCH - 1)], add=True)
        plsc.subcore_barrier()
        pltpu.sync_copy(acc.at[pl.ds(rows0, ROWS_PER_TILE)],
                        out_hbm.at[c, pl.ds(rows0, ROWS_PER_TILE)])

    return _sc_segment_sum


BR = 1000


def _mm_body(x_ref, w_ref, o_ref):
    o_ref[...] = jnp.dot(x_ref[...], w_ref[...],
                         preferred_element_type=jnp.float32)


_mm = pl.pallas_call(
    _mm_body,
    grid=(N // BR,),
    in_specs=[pl.BlockSpec((BR, D), lambda i: (i, 0)),
              pl.BlockSpec((D, D), lambda i: (0, 0))],
    out_specs=pl.BlockSpec((BR, D), lambda i: (i, 0)),
    out_shape=jax.ShapeDtypeStruct((N, D), jnp.float32),
)


def _gru_body(h_ref, m0_ref, m1_ref, Wz_ref, Uz_ref, Wr_ref, Ur_ref,
              Wh_ref, Uh_ref, b_ref, Wn_ref, ho_ref, hwo_ref):
    h = h_ref[...]
    m = m0_ref[...] + m1_ref[...]
    dot = functools.partial(jnp.dot, preferred_element_type=jnp.float32)
    z = jax.nn.sigmoid(dot(m, Wz_ref[...]) + dot(h, Uz_ref[...]) + b_ref[0])
    r = jax.nn.sigmoid(dot(m, Wr_ref[...]) + dot(h, Ur_ref[...]) + b_ref[1])
    hc = jnp.tanh(dot(m, Wh_ref[...]) + dot(r * h, Uh_ref[...]) + b_ref[2])
    hn = (1.0 - z) * h + z * hc
    ho_ref[...] = hn
    hwo_ref[...] = dot(hn, Wn_ref[...])


_gru_step = pl.pallas_call(
    _gru_body,
    grid=(N // BR,),
    in_specs=[pl.BlockSpec((BR, D), lambda i: (i, 0)),
              pl.BlockSpec((BR, D), lambda i: (i, 0)),
              pl.BlockSpec((BR, D), lambda i: (i, 0)),
              pl.BlockSpec((D, D), lambda i: (0, 0)),
              pl.BlockSpec((D, D), lambda i: (0, 0)),
              pl.BlockSpec((D, D), lambda i: (0, 0)),
              pl.BlockSpec((D, D), lambda i: (0, 0)),
              pl.BlockSpec((D, D), lambda i: (0, 0)),
              pl.BlockSpec((D, D), lambda i: (0, 0)),
              pl.BlockSpec((3, D), lambda i: (0, 0)),
              pl.BlockSpec((D, D), lambda i: (0, 0))],
    out_specs=[pl.BlockSpec((BR, D), lambda i: (i, 0)),
               pl.BlockSpec((BR, D), lambda i: (i, 0))],
    out_shape=[jax.ShapeDtypeStruct((N, D), jnp.float32),
               jax.ShapeDtypeStruct((N, D), jnp.float32)],
)


def _shift_down(a):
    return jnp.concatenate([jnp.zeros((1, a.shape[1]), a.dtype), a[:-1]], 0)


def _shift_up(a):
    return jnp.concatenate([a[1:], jnp.zeros((1, a.shape[1]), a.dtype)], 0)


def _conv3_in(a, W_ref, b_ref):
    dot = functools.partial(jnp.dot, preferred_element_type=jnp.float32)
    return (dot(_shift_down(a), W_ref[0]) + dot(a, W_ref[1])
            + dot(_shift_up(a), W_ref[2]) + b_ref[0])


def _conv1_body(h_ref, x_ref, Wa_ref, ba_ref, Wb_ref, bb_ref, ca_ref, cb_ref):
    h = h_ref[...]
    zc = jnp.concatenate([h, x_ref[...]], 1)
    ca_ref[...] = jnp.maximum(_conv3_in(zc, Wa_ref, ba_ref), 0.0)
    cb_ref[...] = jnp.maximum(_conv3_in(h, Wb_ref, bb_ref), 0.0)


_conv1 = pl.pallas_call(
    _conv1_body,
    out_shape=[jax.ShapeDtypeStruct((N, CC), jnp.float32),
               jax.ShapeDtypeStruct((N, CC), jnp.float32)],
)


def _conv2_body(ea_ref, oa_ref, eb_ref, ob_ref, Wa_ref, ba_ref,
                Wb_ref, bb_ref, ca_ref, cb_ref):
    oa = oa_ref[...]
    pa = jnp.maximum(jnp.maximum(ea_ref[...], oa), _shift_down(oa))
    ob = ob_ref[...]
    pb = jnp.maximum(jnp.maximum(eb_ref[...], ob), _shift_down(ob))
    ca_ref[...] = jnp.maximum(_conv3_in(pa, Wa_ref, ba_ref), 0.0)
    cb_ref[...] = jnp.maximum(_conv3_in(pb, Wb_ref, bb_ref), 0.0)


_conv2 = pl.pallas_call(
    _conv2_body,
    out_shape=[jax.ShapeDtypeStruct((N // 2, CC), jnp.float32),
               jax.ShapeDtypeStruct((N // 2, CC), jnp.float32)],
)


def _head_body(ea_ref, oa_ref, eb_ref, ob_ref, wa_ref, ba_ref,
               wb_ref, bb_ref, o_ref):
    dot = functools.partial(jnp.dot, preferred_element_type=jnp.float32)
    oa = oa_ref[...]
    pa = jnp.maximum(jnp.maximum(ea_ref[...], oa), _shift_down(oa))
    ob = ob_ref[...]
    pb = jnp.maximum(jnp.maximum(eb_ref[...], ob), _shift_down(ob))
    ya = dot(pa, wa_ref[...]) + ba_ref[0]
    yb = dot(pb, wb_ref[...]) + bb_ref[0]
    o_ref[...] = jnp.sum(ya * yb, axis=0, keepdims=True) * (1.0 / (N // 4))


_head = pl.pallas_call(
    _head_body,
    out_shape=jax.ShapeDtypeStruct((1, NC), jnp.float32),
)


def kernel(x, ast_edge_index, cfg_edge_index, ddg_edge_index, ncs_edge_index,
           W_edge, Wz, Uz, bz, Wr, Ur, br, Wh, Uh, bh,
           Wa1, ba1, Wa2, ba2, fca_w, fca_b,
           Wb1, bb1, Wb2, bb2, fcb_w, fcb_b):
    edges = [ast_edge_index, cfg_edge_index, ddg_edge_index, ncs_edge_index]
    zeros_tile = jnp.zeros((ROWS_PER_TILE, D), jnp.float32)
    b3 = jnp.stack([bz, br, bh])

    sc_segment_sum = _build_sc_segment_sum()
    h = x
    hw = _mm(h, W_edge[0])
    for t in range(4):
        parts = sc_segment_sum(hw, edges[t][0], edges[t][1], zeros_tile)
        h, hw = _gru_step(h, parts[0, :N], parts[1, :N], Wz, Uz, Wr, Ur, Wh,
                          Uh, b3,
                          W_edge[(t + 1) % 4])

    ca, cb = _conv1(h, x, Wa1, ba1.reshape(1, CC), Wb1, bb1.reshape(1, CC))
    c2a, c2b = _conv2(ca[0::2], ca[1::2], cb[0::2], cb[1::2],
                      Wa2, ba2.reshape(1, CC), Wb2, bb2.reshape(1, CC))
    y = _head(c2a[0::2], c2a[1::2], c2b[0::2], c2b[1::2],
              fca_w, fca_b.reshape(1, NC), fcb_w, fcb_b.reshape(1, NC))
    return y.reshape(NC)
